# baseline (speedup 1.0000x reference)
import functools

import jax
import jax.numpy as jnp
from jax import lax
from jax.experimental import pallas as pl
from jax.experimental.pallas import tpu as pltpu
from jax.experimental.pallas import tpu_sc as plsc


def _make_pane_gather(F, D, V, B):
    info = plsc.get_sparse_core_info()
    NC, NS, L = info.num_cores, info.num_subcores, info.num_lanes
    NW = NC * NS
    P = F * D
    PW = P // NW
    assert P % NW == 0 and D == L
    U = 8
    VA = 49920
    VT = (V // 128) * 128
    NB = VT - VA
    TW = 128
    BSPAN = NB + TW
    CH = 4096
    NCH = B // CH

    mesh = plsc.VectorSubcoreMesh(core_axis_name="c", subcore_axis_name="s")

    @functools.partial(
        pl.kernel,
        mesh=mesh,
        compiler_params=pltpu.CompilerParams(
            use_tc_tiling_on_sc=True, needs_layout_passes=False
        ),
        out_type=jax.ShapeDtypeStruct((P, B), jnp.float32),
        scratch_types=[
            pltpu.VMEM((VA,), jnp.float32),
            pltpu.VMEM((BSPAN,), jnp.float32),
            pltpu.VMEM((B,), jnp.float32),
            pltpu.VMEM((2, CH), jnp.int32),
            pltpu.SemaphoreType.DMA,
            pltpu.SemaphoreType.DMA,
            pltpu.SemaphoreType.DMA,
            pltpu.SemaphoreType.DMA,
            pltpu.SemaphoreType.DMA,
        ],
    )
    def pane_kernel(
        idx_hbm,
        tab_hbm,
        tail_hbm,
        out_hbm,
        pane_a,
        pane_b,
        part_v,
        ix_v,
        sa,
        sb,
        st,
        six,
        so,
    ):
        sid = lax.axis_index("s")
        wid = sid * NC + lax.axis_index("c")
        base = wid * PW


        def fire_a(j):
            return pltpu.async_copy(
                tab_hbm.at[base + j, pl.ds(0, VA)], pane_a, sa
            )

        def fire_b(j):
            return pltpu.async_copy(
                tab_hbm.at[base + j, pl.ds(VA, NB)],
                pane_b.at[pl.ds(0, NB)],
                sb,
            )

        def fire_t(j):
            return pltpu.async_copy(
                tail_hbm.at[base + j], pane_b.at[pl.ds(NB, TW)], st
            )

        def fire_ix(k, slot):
            j, r = k // (2 * NCH), k % NCH
            f = (base + j) // D
            return pltpu.async_copy(
                idx_hbm.at[f, pl.ds(r * CH, CH)], ix_v.at[slot], six
            )

        NK = PW * 2 * NCH

        cp_a, cp_b, cp_t = fire_a(0), fire_b(0), fire_t(0)
        rcp = [fire_ix(0, 0), None]
        cur = 0
        o_cps = []
        for j in range(PW):
            p = base + j
            cp_a.wait()
            for oc in o_cps:
                oc.wait()
            o_cps = []
            for c in range(NCH):
                k = j * 2 * NCH + c
                rcp[cur].wait()
                if k + 1 < NK:
                    rcp[1 - cur] = fire_ix(k + 1, 1 - cur)
                sl = cur

                def blk_a(i, carry):
                    b0 = i * L * U
                    for u in range(U):
                        iv = ix_v[sl, pl.ds(b0 + u * L, L)]
                        a = jnp.minimum(iv, VA - 1)
                        part_v[pl.ds(c * CH + b0 + u * L, L)] = (
                            plsc.load_gather(pane_a, [a])
                        )
                    return carry

                lax.fori_loop(0, CH // (L * U), blk_a, 0)
                cur = 1 - cur
            if j + 1 < PW:
                cp_a = fire_a(j + 1)
            cp_b.wait()
            cp_t.wait()
            for c in range(NCH):
                k = j * 2 * NCH + NCH + c
                rcp[cur].wait()
                if k + 1 < NK:
                    rcp[1 - cur] = fire_ix(k + 1, 1 - cur)
                sl = cur

                def blk_b(i, carry):
                    b0 = i * L * U
                    for u in range(U):
                        pos = pl.ds(c * CH + b0 + u * L, L)
                        iv = ix_v[sl, pl.ds(b0 + u * L, L)]
                        a = jnp.maximum(iv - VA, 0)
                        r2 = plsc.load_gather(pane_b, [a])
                        part_v[pos] = jnp.where(iv < VA, part_v[pos], r2)
                    return carry

                lax.fori_loop(0, CH // (L * U), blk_b, 0)
                o_cps.append(
                    pltpu.async_copy(
                        part_v.at[pl.ds(c * CH, CH)],
                        out_hbm.at[p, pl.ds(c * CH, CH)],
                        so,
                    )
                )
                cur = 1 - cur
            if j + 1 < PW:
                cp_b, cp_t = fire_b(j + 1), fire_t(j + 1)
        for oc in o_cps:
            oc.wait()

    return pane_kernel


def kernel(indices, tables):
    B, F = indices.shape
    _, V, D = tables.shape
    idx_t = indices.T
    tab_panes = tables.transpose(0, 2, 1).reshape(F * D, V)
    VT = (V // 128) * 128
    tail = jnp.pad(tab_panes[:, VT:], ((0, 0), (0, 128 - (V - VT))))
    out_t = _make_pane_gather(F, D, V, B)(idx_t, tab_panes, tail)
    return out_t.reshape(F, D, B).transpose(2, 0, 1)


# speedup vs baseline: 2.8788x; 2.8788x over previous
import functools

import jax
import jax.numpy as jnp
from jax import lax
from jax.experimental import pallas as pl
from jax.experimental.pallas import tpu as pltpu
from jax.experimental.pallas import tpu_sc as plsc


def _make_pane_gather(F, D, V, B):
    info = plsc.get_sparse_core_info()
    NC, NS, L = info.num_cores, info.num_subcores, info.num_lanes
    NW = NC * NS
    P = F * D
    PW = P // NW
    assert P % NW == 0 and D == L
    QB = B // 4
    U = 8
    VA = (V // 2) // 128 * 128
    VT = (V // 128) * 128
    TW = 128

    mesh = plsc.VectorSubcoreMesh(core_axis_name="c", subcore_axis_name="s")

    @functools.partial(
        pl.kernel,
        mesh=mesh,
        compiler_params=pltpu.CompilerParams(
            use_tc_tiling_on_sc=True, needs_layout_passes=False
        ),
        out_type=jax.ShapeDtypeStruct((P, B), jnp.float32),
        scratch_types=[
            pltpu.VMEM((VT + TW,), jnp.float32),
            pltpu.VMEM((B,), jnp.int32),
            pltpu.VMEM((2, QB), jnp.float32),
            pltpu.SemaphoreType.DMA,
            pltpu.SemaphoreType.DMA,
        ],
    )
    def pane_kernel(
        idx_hbm, tab_hbm, tail_hbm, out_hbm, pane_v, idx_v, out_v, psem, osem
    ):
        wid = lax.axis_index("s") * NC + lax.axis_index("c")

        o_cp = [None, None]
        for j in range(PW):
            p = wid * PW + j
            f = p // D
            pane_cps = [
                pltpu.async_copy(
                    tab_hbm.at[p, pl.ds(0, VA)], pane_v.at[pl.ds(0, VA)], psem
                ),
                pltpu.async_copy(
                    tab_hbm.at[p, pl.ds(VA, VT - VA)],
                    pane_v.at[pl.ds(VA, VT - VA)],
                    psem,
                ),
                pltpu.async_copy(
                    tail_hbm.at[p], pane_v.at[pl.ds(VT, TW)], psem
                ),
            ]
            @pl.when(jnp.logical_or(p % D == 0, j == 0))
            def _load_idx():
                pltpu.sync_copy(idx_hbm.at[f, pl.ds(0, B)], idx_v)
            for cp in pane_cps:
                cp.wait()

            for q in range(4):
                s = q % 2

                def gather_block(i, carry):
                    b0 = i * L * U
                    ivs = [
                        idx_v[pl.ds(q * QB + b0 + k * L, L)] for k in range(U)
                    ]
                    res = [plsc.load_gather(pane_v, [iv]) for iv in ivs]
                    for k in range(U):
                        out_v[s, pl.ds(b0 + k * L, L)] = res[k]
                    return carry

                if o_cp[s] is not None:
                    o_cp[s].wait()
                lax.fori_loop(0, QB // (L * U), gather_block, 0)
                o_cp[s] = pltpu.async_copy(
                    out_v.at[s], out_hbm.at[p, pl.ds(q * QB, QB)], osem
                )
        o_cp[0].wait()
        o_cp[1].wait()

    return pane_kernel


def kernel(indices, tables):
    B, F = indices.shape
    _, V, D = tables.shape
    idx_t = indices.T
    tab_panes = tables.transpose(0, 2, 1).reshape(F * D, V)
    VT = (V // 128) * 128
    tail = jnp.pad(tab_panes[:, VT:], ((0, 0), (0, 128 - (V - VT))))
    out_t = _make_pane_gather(F, D, V, B)(idx_t, tab_panes, tail)
    return out_t.reshape(F, D, B).transpose(2, 0, 1)


# speedup vs baseline: 2.9140x; 1.0122x over previous
import functools

import jax
import jax.numpy as jnp
from jax import lax
from jax.experimental import pallas as pl
from jax.experimental.pallas import tpu as pltpu
from jax.experimental.pallas import tpu_sc as plsc


def _make_pane_gather(F, D, V, B):
    info = plsc.get_sparse_core_info()
    NC, NS, L = info.num_cores, info.num_subcores, info.num_lanes
    NW = NC * NS
    P = F * D
    PW = P // NW
    assert P % NW == 0 and D == L
    QB = B // 4
    U = 8

    mesh = plsc.VectorSubcoreMesh(core_axis_name="c", subcore_axis_name="s")

    @functools.partial(
        pl.kernel,
        mesh=mesh,
        compiler_params=pltpu.CompilerParams(
            use_tc_tiling_on_sc=True, needs_layout_passes=False
        ),
        out_type=jax.ShapeDtypeStruct((P, B), jnp.float32),
        scratch_types=[
            pltpu.VMEM((V,), jnp.float32),
            pltpu.VMEM((B,), jnp.int32),
            pltpu.VMEM((2, QB), jnp.float32),
            pltpu.SemaphoreType.DMA,
            pltpu.SemaphoreType.DMA,
        ],
    )
    def pane_kernel(idx_hbm, tab_hbm, out_hbm, pane_v, idx_v, out_v, psem, osem):
        wid = lax.axis_index("s") * NC + lax.axis_index("c")

        o_cp = [None, None]
        for j in range(PW):
            p = wid * PW + j
            f = p // D
            pane_cps = [pltpu.async_copy(tab_hbm.at[p], pane_v, psem)]
            @pl.when(jnp.logical_or(p % D == 0, j == 0))
            def _load_idx():
                pltpu.sync_copy(idx_hbm.at[f, pl.ds(0, B)], idx_v)
            for cp in pane_cps:
                cp.wait()

            for q in range(4):
                s = q % 2

                def gather_block(i, carry):
                    b0 = i * L * U
                    ivs = [
                        idx_v[pl.ds(q * QB + b0 + k * L, L)] for k in range(U)
                    ]
                    res = [plsc.load_gather(pane_v, [iv]) for iv in ivs]
                    for k in range(U):
                        out_v[s, pl.ds(b0 + k * L, L)] = res[k]
                    return carry

                if o_cp[s] is not None:
                    o_cp[s].wait()
                lax.fori_loop(0, QB // (L * U), gather_block, 0)
                o_cp[s] = pltpu.async_copy(
                    out_v.at[s], out_hbm.at[p, pl.ds(q * QB, QB)], osem
                )
        o_cp[0].wait()
        o_cp[1].wait()

    return pane_kernel


def kernel(indices, tables):
    B, F = indices.shape
    _, V, D = tables.shape
    idx_t = indices.T
    tab_panes = tables.transpose(0, 2, 1).reshape(F * D, V)
    out_t = _make_pane_gather(F, D, V, B)(idx_t, tab_panes)
    return out_t.reshape(F, D, B).transpose(2, 0, 1)
